# Initial kernel scaffold; baseline (speedup 1.0000x reference)
import jax
import jax.numpy as jnp
from jax.experimental import pallas as pl


def kernel(x, rels, edge_index, edge_type, w_in1, w_out1, w_loop1, w_rel1, w_in2, w_out2, w_loop2, w_rel2, loop_rel1, loop_rel2):
    raise NotImplementedError("write your pallas kernel here")


# speedup vs baseline: 4.1698x; 4.1698x over previous
import functools

import jax
import jax.numpy as jnp
from jax import lax
from jax.experimental import pallas as pl
from jax.experimental.pallas import tpu as pltpu
from jax.experimental.pallas import tpu_sc as plsc

N = 10000
NP = 10240
D = 256
NR2 = 400
NE_HALF = 80000
EP = 81920
NT = 16
NCH = 20
CK = 256
ROWS_PT = NP // NT
CROWS = NP * NR2 // 128
CQROWS = CROWS // 4
CPADR = 128
CACC_R = CQROWS + CPADR
CZ_R = 512
CF_R = 504

@functools.cache
def _mesh():
    return plsc.VectorSubcoreMesh(core_axis_name="c", subcore_axis_name="s")


def _sc_deg_body(deg_src, ones_hbm, z2d, deg_out, idx_v, ones_v, dacc, sem):
    c = lax.axis_index("c")
    s = lax.axis_index("s")
    pltpu.sync_copy(z2d, dacc.at[pl.ds(s * ROWS_PT, ROWS_PT)])
    pltpu.sync_copy(ones_hbm, ones_v)
    plsc.subcore_barrier()
    base = (c * NT + s) * NCH * CK

    def chunk(j, carry):
        pltpu.sync_copy(deg_src.at[pl.ds(base + j * CK, CK)], idx_v)
        pltpu.sync_copy(ones_v, dacc.at[idx_v], add=True)
        return carry

    lax.fori_loop(0, NCH, chunk, 0)
    plsc.subcore_barrier()
    pltpu.sync_copy(dacc.at[pl.ds(s * ROWS_PT, ROWS_PT)],
                    deg_out.at[pl.ds(c * NP + s * ROWS_PT, ROWS_PT)])


def _sc_deg(deg_src, ones_hbm, z2d):
    return pl.kernel(
        _sc_deg_body,
        out_type=jax.ShapeDtypeStruct((2 * NP, 128), jnp.float32),
        mesh=_mesh(),
        scratch_types=[
            pltpu.VMEM((CK,), jnp.int32),
            pltpu.VMEM((CK, 128), jnp.float32),
            pltpu.VMEM_SHARED((NP, 128), jnp.float32),
            pltpu.SemaphoreType.DMA,
        ],
    )(deg_src, ones_hbm, z2d)


def _sc_dval_body(dinv_t, didx, dval_out, idx_v0, idx_v1,
                  rows_v0, rows_v1, sem0, sem1):
    c = lax.axis_index("c")
    s = lax.axis_index("s")
    base = c * EP + s * NCH * CK
    pltpu.sync_copy(didx.at[pl.ds(base, CK)], idx_v0)
    pltpu.async_copy(dinv_t.at[idx_v0], rows_v0, sem0)

    def pair(jj, carry):
        j0 = 2 * jj
        pltpu.sync_copy(didx.at[pl.ds(base + (j0 + 1) * CK, CK)], idx_v1)
        pltpu.async_copy(dinv_t.at[idx_v1], rows_v1, sem1)
        pltpu.make_async_copy(dinv_t.at[idx_v0], rows_v0, sem0).wait()
        pltpu.sync_copy(rows_v0, dval_out.at[pl.ds(base + j0 * CK, CK)])

        @pl.when(jj < NCH // 2 - 1)
        def _():
            pltpu.sync_copy(didx.at[pl.ds(base + (j0 + 2) * CK, CK)], idx_v0)
            pltpu.async_copy(dinv_t.at[idx_v0], rows_v0, sem0)

        pltpu.make_async_copy(dinv_t.at[idx_v1], rows_v1, sem1).wait()
        pltpu.sync_copy(rows_v1, dval_out.at[pl.ds(base + (j0 + 1) * CK, CK)])
        return carry

    lax.fori_loop(0, NCH // 2, pair, 0)


def _sc_dval(dinv_t, didx):
    return pl.kernel(
        _sc_dval_body,
        out_type=jax.ShapeDtypeStruct((2 * EP, 128), jnp.float32),
        mesh=_mesh(),
        scratch_types=[
            pltpu.VMEM((CK,), jnp.int32),
            pltpu.VMEM((CK,), jnp.int32),
            pltpu.VMEM((CK, 128), jnp.float32),
            pltpu.VMEM((CK, 128), jnp.float32),
            pltpu.SemaphoreType.DMA,
            pltpu.SemaphoreType.DMA,
        ],
    )(dinv_t, didx)


def _sc_cbuild_body(rowidx, oh, zcb, cin_out, cout_out,
                    rowb, rows_v0, rows_v1, cacc, sem0, sem1):
    c = lax.axis_index("c")
    s = lax.axis_index("s")
    for d, out in ((0, cin_out), (1, cout_out)):
        for p in (0, 1):
            q = 2 * p + c

            @pl.when(s < NT - 1)
            def _():
                pltpu.sync_copy(zcb, cacc.at[pl.ds(s * CZ_R, CZ_R)])

            @pl.when(s == NT - 1)
            def _():
                pltpu.sync_copy(zcb.at[pl.ds(0, CACC_R - (NT - 1) * CZ_R)],
                                cacc.at[pl.ds((NT - 1) * CZ_R,
                                              CACC_R - (NT - 1) * CZ_R)])

            plsc.subcore_barrier()
            ribase = (d * 4 + q) * EP
            ohbase = d * EP
            tbase = s * NCH * CK

            pltpu.async_copy(oh.at[pl.ds(ohbase + tbase, CK)], rows_v0, sem0)

            def pair(jj, carry):
                j0 = 2 * jj
                pltpu.async_copy(
                    oh.at[pl.ds(ohbase + tbase + (j0 + 1) * CK, CK)],
                    rows_v1, sem1)
                pltpu.make_async_copy(
                    oh.at[pl.ds(ohbase + tbase + j0 * CK, CK)],
                    rows_v0, sem0).wait()
                pltpu.sync_copy(rowidx.at[pl.ds(ribase + tbase + j0 * CK, CK)],
                                rowb)
                pltpu.sync_copy(
                    rows_v0,
                    cacc.at[plsc.Indices(rowb, ignored_value=-1)], add=True)

                @pl.when(jj < NCH // 2 - 1)
                def _():
                    pltpu.async_copy(
                        oh.at[pl.ds(ohbase + tbase + (j0 + 2) * CK, CK)],
                        rows_v0, sem0)

                pltpu.make_async_copy(
                    oh.at[pl.ds(ohbase + tbase + (j0 + 1) * CK, CK)],
                    rows_v1, sem1).wait()
                pltpu.sync_copy(
                    rowidx.at[pl.ds(ribase + tbase + (j0 + 1) * CK, CK)],
                    rowb)
                pltpu.sync_copy(
                    rows_v1,
                    cacc.at[plsc.Indices(rowb, ignored_value=-1)], add=True)
                return carry

            lax.fori_loop(0, NCH // 2, pair, 0)
            plsc.subcore_barrier()

            @pl.when(s < NT - 1)
            def _():
                pltpu.sync_copy(
                    cacc.at[pl.ds(s * CF_R, CF_R)],
                    out.at[pl.ds(q * CQROWS + s * CF_R, CF_R)])

            @pl.when(s == NT - 1)
            def _():
                rem = CQROWS - (NT - 1) * CF_R
                pltpu.sync_copy(
                    cacc.at[pl.ds((NT - 1) * CF_R, rem)],
                    out.at[pl.ds(q * CQROWS + (NT - 1) * CF_R, rem)])

            plsc.subcore_barrier()


def _sc_cbuild(rowidx, oh, zcb):
    return pl.kernel(
        _sc_cbuild_body,
        out_type=(jax.ShapeDtypeStruct((CROWS, 128), jnp.float32),
                  jax.ShapeDtypeStruct((CROWS, 128), jnp.float32)),
        mesh=_mesh(),
        scratch_types=[
            pltpu.VMEM((CK,), jnp.int32),
            pltpu.VMEM((CK, 128), jnp.float32),
            pltpu.VMEM((CK, 128), jnp.float32),
            pltpu.VMEM_SHARED((CACC_R, 128), jnp.float32),
            pltpu.SemaphoreType.DMA,
            pltpu.SemaphoreType.DMA,
        ],
    )(rowidx, oh, zcb)


AGG_CK = 128
AGG_NCH = (NCH * CK) // AGG_CK


def _sc_agg_body(u_in0, u_in1, u_out0, u_out1, src_in, dst_in, src_out,
                 dst_out, z2d, a_out, src_v0, src_v1, dst_v, rows_v0, rows_v1,
                 acc, sem0, sem1):
    c = lax.axis_index("c")
    s = lax.axis_index("s")
    for d, tabs, srcs, dsts in ((0, (u_in0, u_in1), src_in, dst_in),
                                (1, (u_out0, u_out1), src_out, dst_out)):
        pltpu.sync_copy(z2d, acc.at[pl.ds(s * ROWS_PT, ROWS_PT)])
        plsc.subcore_barrier()
        base = s * AGG_NCH * AGG_CK

        def run(u):
            pltpu.sync_copy(srcs.at[pl.ds(base, AGG_CK)], src_v0)
            pltpu.async_copy(u.at[src_v0], rows_v0, sem0)

            def pair(jj, carry):
                j0 = 2 * jj
                pltpu.sync_copy(
                    srcs.at[pl.ds(base + (j0 + 1) * AGG_CK, AGG_CK)], src_v1)
                pltpu.async_copy(u.at[src_v1], rows_v1, sem1)
                pltpu.make_async_copy(u.at[src_v0], rows_v0, sem0).wait()
                pltpu.sync_copy(dsts.at[pl.ds(base + j0 * AGG_CK, AGG_CK)],
                                dst_v)
                pltpu.sync_copy(rows_v0, acc.at[dst_v], add=True)

                @pl.when(jj < AGG_NCH // 2 - 1)
                def _():
                    pltpu.sync_copy(
                        srcs.at[pl.ds(base + (j0 + 2) * AGG_CK, AGG_CK)],
                        src_v0)
                    pltpu.async_copy(u.at[src_v0], rows_v0, sem0)

                pltpu.make_async_copy(u.at[src_v1], rows_v1, sem1).wait()
                pltpu.sync_copy(
                    dsts.at[pl.ds(base + (j0 + 1) * AGG_CK, AGG_CK)], dst_v)
                pltpu.sync_copy(rows_v1, acc.at[dst_v], add=True)
                return carry

            lax.fori_loop(0, AGG_NCH // 2, pair, 0)

        @pl.when(c == 0)
        def _():
            run(tabs[0])

        @pl.when(c == 1)
        def _():
            run(tabs[1])

        plsc.subcore_barrier()
        pltpu.sync_copy(
            acc.at[pl.ds(s * ROWS_PT, ROWS_PT)],
            a_out.at[pl.ds((d * 2 + c) * NP + s * ROWS_PT, ROWS_PT)])
        plsc.subcore_barrier()


def _sc_agg(u_in0, u_in1, u_out0, u_out1, src_in, dst_in, src_out, dst_out,
            z2d):
    return pl.kernel(
        _sc_agg_body,
        out_type=jax.ShapeDtypeStruct((4 * NP, 128), jnp.float32),
        mesh=_mesh(),
        scratch_types=[
            pltpu.VMEM((AGG_CK,), jnp.int32),
            pltpu.VMEM((AGG_CK,), jnp.int32),
            pltpu.VMEM((AGG_CK,), jnp.int32),
            pltpu.VMEM((AGG_CK, 128), jnp.float32),
            pltpu.VMEM((AGG_CK, 128), jnp.float32),
            pltpu.VMEM_SHARED((NP, 128), jnp.float32),
            pltpu.SemaphoreType.DMA,
            pltpu.SemaphoreType.DMA,
        ],
    )(u_in0, u_in1, u_out0, u_out1, src_in, dst_in, src_out, dst_out, z2d)


def _dinv_body(deg_ref, o_ref, ot_ref):
    deg = deg_ref[:, 0:1]
    dv = jnp.where(deg > 0, lax.rsqrt(deg), 0.0)
    o_ref[...] = dv.reshape(2, NP)
    ot_ref[...] = jnp.broadcast_to(dv, (2 * NP, 128))


def _tc_dinv(deg):
    return pl.pallas_call(
        _dinv_body,
        out_shape=[jax.ShapeDtypeStruct((2, NP), jnp.float32),
                   jax.ShapeDtypeStruct((2 * NP, 128), jnp.float32)],
    )(deg)


def _oh_body(dval_ref, lane_ref, o_ref):
    iota = lax.broadcasted_iota(jnp.int32, (dval_ref.shape[0], 128), 1)
    o_ref[...] = jnp.where(iota == lane_ref[...], dval_ref[...], 0.0)


def _tc_oh(dval1, lane1):
    bm = 2048
    return pl.pallas_call(
        _oh_body,
        grid=(2 * EP // bm,),
        in_specs=[
            pl.BlockSpec((bm, 1), lambda i: (i, 0)),
            pl.BlockSpec((bm, 1), lambda i: (i, 0)),
        ],
        out_specs=pl.BlockSpec((bm, 128), lambda i: (i, 0)),
        out_shape=jax.ShapeDtypeStruct((2 * EP, 128), jnp.float32),
    )(dval1, lane1)


def _split_out(h, dv_ref, t0, t1, t2, t3, ml):
    din = dv_ref[:, 0:1]
    dout = dv_ref[:, 1:2]
    t0[...] = h[:, 0:128] * din
    t1[...] = h[:, 128:256] * din
    t2[...] = h[:, 256:384] * dout
    t3[...] = h[:, 384:512] * dout
    ml[...] = h[:, 512:768]


_MM_OUT = [jax.ShapeDtypeStruct((NP, 128), jnp.float32)] * 4 + [
    jax.ShapeDtypeStruct((NP, 256), jnp.float32)]


def _mm_out_specs():
    return [pl.BlockSpec((512, 128), lambda i: (i, 0))] * 4 + [
        pl.BlockSpec((512, 256), lambda i: (i, 0))]


def _mm_scaled_body(x_ref, w_ref, dv_ref, t0, t1, t2, t3, ml):
    h = jnp.dot(x_ref[...], w_ref[...], preferred_element_type=jnp.float32)
    _split_out(h, dv_ref, t0, t1, t2, t3, ml)


def _tc_mm_scaled(x, w3, dinv2):
    bm = 512
    return pl.pallas_call(
        _mm_scaled_body,
        grid=(NP // bm,),
        in_specs=[
            pl.BlockSpec((bm, D), lambda i: (i, 0)),
            pl.BlockSpec((D, 768), lambda i: (0, 0)),
            pl.BlockSpec((bm, 2), lambda i: (i, 0)),
        ],
        out_specs=_mm_out_specs(),
        out_shape=list(_MM_OUT),
    )(x, w3, dinv2)


def _mm_norm_scaled_body(pre_ref, st_ref, w_ref, dv_ref, t0, t1, t2, t3, ml):
    i = pl.program_id(0)
    m = st_ref[0:1, :] * (1.0 / N)
    var = st_ref[1:2, :] * (1.0 / N) - m * m
    rs = lax.rsqrt(var + 1e-5)
    xa = jnp.maximum((pre_ref[...] - m) * rs, 0.0)
    rows = i * pre_ref.shape[0] + lax.broadcasted_iota(
        jnp.int32, (pre_ref.shape[0], 1), 0)
    xa = jnp.where(rows < N, xa, 0.0)
    h = jnp.dot(xa, w_ref[...], preferred_element_type=jnp.float32)
    _split_out(h, dv_ref, t0, t1, t2, t3, ml)


def _tc_mm_norm_scaled(pre, stats, w3, dinv2):
    bm = 512
    return pl.pallas_call(
        _mm_norm_scaled_body,
        grid=(NP // bm,),
        in_specs=[
            pl.BlockSpec((bm, D), lambda i: (i, 0)),
            pl.BlockSpec((2, D), lambda i: (0, 0)),
            pl.BlockSpec((D, 768), lambda i: (0, 0)),
            pl.BlockSpec((bm, 2), lambda i: (i, 0)),
        ],
        out_specs=_mm_out_specs(),
        out_shape=list(_MM_OUT),
    )(pre, stats, w3, dinv2)


def _mm_small_body(a_ref, b_ref, o_ref):
    o_ref[...] = jnp.dot(a_ref[...], b_ref[...],
                         preferred_element_type=jnp.float32)


def _tc_mm_small(a, b):
    return pl.pallas_call(
        _mm_small_body,
        out_shape=jax.ShapeDtypeStruct((a.shape[0], b.shape[1]), jnp.float32),
    )(a, b)


def _tc_mm(a, b):
    bm = 512
    m, k = a.shape
    n = b.shape[1]
    return pl.pallas_call(
        _mm_small_body,
        grid=(m // bm,),
        in_specs=[
            pl.BlockSpec((bm, k), lambda i: (i, 0)),
            pl.BlockSpec((k, n), lambda i: (0, 0)),
        ],
        out_specs=pl.BlockSpec((bm, n), lambda i: (i, 0)),
        out_shape=jax.ShapeDtypeStruct((m, n), jnp.float32),
    )(a, b)


def _combine_body(ain_ref, aout_ref, bin_ref, bout_ref, ml_ref, lr_ref, dv_ref,
                  pre_ref, st_ref):
    i = pl.program_id(1)
    din = dv_ref[:, 0:1]
    dout = dv_ref[:, 1:2]
    pre = (din * (ain_ref[0, 0] - bin_ref[...])
           + dout * (aout_ref[0, 0] - bout_ref[...])
           + (ml_ref[...] - lr_ref[...])) * (1.0 / 3.0)
    pre_ref[...] = pre
    bm = pre.shape[0]
    rows = i * bm + lax.broadcasted_iota(jnp.int32, (bm, 1), 0)
    prem = jnp.where(rows < N, pre, 0.0)
    s1 = jnp.sum(prem, axis=0, keepdims=True)
    s2 = jnp.sum(prem * prem, axis=0, keepdims=True)
    st = jnp.concatenate([s1, s2], axis=0)

    @pl.when(i == 0)
    def _():
        st_ref[...] = st

    @pl.when(i > 0)
    def _():
        st_ref[...] += st


def _tc_combine(a4, b_in, b_out, mloop, lr, dinv2):
    bm = 128
    grid = (2, NP // bm)
    return pl.pallas_call(
        _combine_body,
        grid=grid,
        in_specs=[
            pl.BlockSpec((1, 1, bm, 128), lambda j, i: (0, j, i, 0)),
            pl.BlockSpec((1, 1, bm, 128), lambda j, i: (1, j, i, 0)),
            pl.BlockSpec((bm, 128), lambda j, i: (i, j)),
            pl.BlockSpec((bm, 128), lambda j, i: (i, j)),
            pl.BlockSpec((bm, 128), lambda j, i: (i, j)),
            pl.BlockSpec((1, 128), lambda j, i: (0, j)),
            pl.BlockSpec((bm, 2), lambda j, i: (i, 0)),
        ],
        out_specs=[
            pl.BlockSpec((bm, 128), lambda j, i: (i, j)),
            pl.BlockSpec((2, 128), lambda j, i: (0, j)),
        ],
        out_shape=[
            jax.ShapeDtypeStruct((NP, D), jnp.float32),
            jax.ShapeDtypeStruct((2, D), jnp.float32),
        ],
    )(a4, a4, b_in, b_out, mloop, lr, dinv2)


def _norm_body(pre_ref, st_ref, o_ref):
    m = st_ref[0:1, :] * (1.0 / N)
    var = st_ref[1:2, :] * (1.0 / N) - m * m
    rs = lax.rsqrt(var + 1e-5)
    o_ref[...] = jnp.maximum((pre_ref[...] - m) * rs, 0.0)


def _tc_normalize(pre, stats):
    bm = 400
    return pl.pallas_call(
        _norm_body,
        grid=(N // bm,),
        in_specs=[
            pl.BlockSpec((bm, D), lambda i: (i, 0)),
            pl.BlockSpec((2, D), lambda i: (0, 0)),
        ],
        out_specs=pl.BlockSpec((bm, D), lambda i: (i, 0)),
        out_shape=jax.ShapeDtypeStruct((N, D), jnp.float32),
    )(pre, stats)


def kernel(x, rels, edge_index, edge_type, w_in1, w_out1, w_loop1, w_rel1,
           w_in2, w_out2, w_loop2, w_rel2, loop_rel1, loop_rel2):
    ne = NE_HALF
    pad_e = EP - ne
    ei = edge_index.astype(jnp.int32)
    et = edge_type.astype(jnp.int32)
    pad_rows = N + (jnp.arange(pad_e, dtype=jnp.int32) % (NP - N))

    def prep(src, dst, t):
        srcp = jnp.concatenate([src, pad_rows])
        dstp = jnp.concatenate([dst, pad_rows])
        tp = jnp.concatenate([t, jnp.zeros((pad_e,), jnp.int32)])
        flat = dstp * NR2 + tp
        lane = flat % 128
        rfull = flat // 128
        ridx = []
        for q in range(4):
            loc = rfull - q * CQROWS
            ok = (loc >= 0) & (loc < CQROWS)
            ridx.append(jnp.where(ok, loc, -1))
        return srcp, dstp, lane, jnp.concatenate(ridx)

    in_srcp, in_dstp, in_lane, in_ridx = prep(
        ei[0, :ne], ei[1, :ne], et[:ne])
    out_srcp, out_dstp, out_lane, out_ridx = prep(
        ei[0, ne:], ei[1, ne:], et[ne:])

    ones_hbm = jnp.ones((CK, 128), jnp.float32)
    z2d = jnp.zeros((ROWS_PT, 128), jnp.float32)
    zcb = jnp.zeros((CZ_R, 128), jnp.float32)

    deg_src = jnp.concatenate([in_srcp, out_srcp])
    deg = _sc_deg(deg_src, ones_hbm, z2d)
    dinv, dinv_t = _tc_dinv(deg)
    dinv2 = dinv.T

    didx = jnp.concatenate([in_srcp, out_srcp + NP])
    dval = _sc_dval(dinv_t, didx)
    lane1 = jnp.concatenate([in_lane, out_lane]).reshape(2 * EP, 1)
    oh = _tc_oh(dval[:, 0:1], lane1)
    rowidx = jnp.concatenate([in_ridx, out_ridx])
    cin_r, cout_r = _sc_cbuild(rowidx, oh, zcb)
    c_in = cin_r.reshape(NP, NR2)
    c_out = cout_r.reshape(NP, NR2)

    x_pad = jnp.pad(x, ((0, NP - N), (0, 0)))

    def layer(big_mm, big_args, rel_table, loop_rel, w_in, w_out, w_loop, w_rel):
        w4 = jnp.concatenate([w_in, w_out, w_loop, w_rel], axis=1)
        t_in0, t_in1, t_out0, t_out1, mloop = big_mm(*big_args, w4[:, :768],
                                                     dinv2)
        ra = jnp.concatenate([rel_table, loop_rel], axis=0)
        ra = jnp.pad(ra, ((0, 512 - ra.shape[0]), (0, 0)))
        r4 = _tc_mm_small(ra, w4)
        rw_in = r4[:NR2, 0:256]
        rw_out = r4[:NR2, 256:512]
        lr = r4[NR2:NR2 + 1, 512:768]
        r_next = r4[:NR2, 768:1024]
        a4 = _sc_agg(t_in0, t_in1, t_out0, t_out1, in_srcp, in_dstp,
                     out_srcp, out_dstp, z2d).reshape(2, 2, NP, 128)
        b_in = _tc_mm(c_in, rw_in)
        b_out = _tc_mm(c_out, rw_out)
        pre, stats = _tc_combine(a4, b_in, b_out, mloop, lr, dinv2)
        return pre, stats, r_next

    pre1, st1, r1 = layer(_tc_mm_scaled, (x_pad,), rels, loop_rel1,
                          w_in1, w_out1, w_loop1, w_rel1)
    pre2, st2, r2 = layer(_tc_mm_norm_scaled, (pre1, st1), r1, loop_rel2,
                          w_in2, w_out2, w_loop2, w_rel2)
    x2 = _tc_normalize(pre2, st2)
    return (x2, r2)
